# baseline (speedup 1.0000x reference)
import functools

import jax
import jax.numpy as jnp
from jax import lax
from jax.experimental import pallas as pl
from jax.experimental.pallas import tpu as pltpu
from jax.experimental.pallas import tpu_sc as plsc

NC = 2
NS = 16
NW = NC * NS
CHUNK = 128


@functools.partial(jax.jit, static_argnames=("nchunk", "dim"))
def _gather_sc(x_flat, weight, nchunk, dim):
    b_total = x_flat.shape[0]
    idx3 = x_flat.reshape(NW, nchunk, CHUNK)
    mesh = plsc.VectorSubcoreMesh(
        core_axis_name="c", subcore_axis_name="s", num_cores=NC, num_subcores=NS
    )

    @functools.partial(
        pl.kernel,
        out_type=jax.ShapeDtypeStruct((b_total, dim), jnp.float32),
        mesh=mesh,
        scratch_types=[
            pltpu.VMEM((nchunk, CHUNK), jnp.int32),
            pltpu.VMEM((CHUNK, dim), jnp.float32),
            pltpu.SemaphoreType.DMA,
        ],
        compiler_params=pltpu.CompilerParams(use_tc_tiling_on_sc=False),
    )
    def k(idx_hbm, table_hbm, out_hbm, idx_v, rows_v, sem):
        wid = lax.axis_index("s") * NC + lax.axis_index("c")
        pltpu.sync_copy(idx_hbm.at[wid], idx_v)
        base = wid * (nchunk * CHUNK)

        def body(c, carry):
            pltpu.async_copy(table_hbm.at[idx_v.at[c]], rows_v, sem).wait()
            pltpu.sync_copy(rows_v, out_hbm.at[pl.ds(base + c * CHUNK, CHUNK)])
            return carry

        lax.fori_loop(0, nchunk, body, 0)

    return k(idx3, weight)


def kernel(x, weight):
    dim = weight.shape[1]
    b_total = x.size
    nchunk = b_total // (NW * CHUNK)
    out = _gather_sc(x.reshape(-1), weight, nchunk, dim)
    return out.reshape(x.shape + (dim,))


# speedup vs baseline: 1.0473x; 1.0473x over previous
import functools

import jax
import jax.numpy as jnp
from jax import lax
from jax.experimental import pallas as pl
from jax.experimental.pallas import tpu as pltpu
from jax.experimental.pallas import tpu_sc as plsc

NC = 2
NS = 16
NW = NC * NS
CHUNK = 128
LOOKAHEAD = 5
NBUF = 10


@functools.partial(jax.jit, static_argnames=("nchunk", "dim"))
def _gather_sc(x_flat, weight, nchunk, dim):
    b_total = x_flat.shape[0]
    idx3 = x_flat.reshape(NW, nchunk, CHUNK)
    mesh = plsc.VectorSubcoreMesh(
        core_axis_name="c", subcore_axis_name="s", num_cores=NC, num_subcores=NS
    )

    @functools.partial(
        pl.kernel,
        out_type=jax.ShapeDtypeStruct((b_total, dim), jnp.float32),
        mesh=mesh,
        scratch_types=[
            pltpu.VMEM((nchunk, CHUNK), jnp.int32),
            pltpu.VMEM((NBUF, CHUNK, dim), jnp.float32),
            pltpu.SemaphoreType.DMA((NBUF,)),
            pltpu.SemaphoreType.DMA((NBUF,)),
        ],
        compiler_params=pltpu.CompilerParams(use_tc_tiling_on_sc=False),
    )
    def k(idx_hbm, table_hbm, out_hbm, idx_v, rows_v, gsem, wsem):
        wid = lax.axis_index("s") * NC + lax.axis_index("c")
        pltpu.sync_copy(idx_hbm.at[wid], idx_v)
        base = wid * (nchunk * CHUNK)

        def issue_gather(c):
            b = c % NBUF
            return pltpu.async_copy(
                table_hbm.at[idx_v.at[c]], rows_v.at[b], gsem.at[b]
            )

        def issue_write(c):
            b = c % NBUF
            return pltpu.async_copy(
                rows_v.at[b], out_hbm.at[pl.ds(base + c * CHUNK, CHUNK)], wsem.at[b]
            )

        gathers, writes = {}, {}
        for c in range(min(LOOKAHEAD, nchunk)):
            gathers[c] = issue_gather(c)
        for j in range(nchunk):
            f = j + LOOKAHEAD
            if f < nchunk:
                if f >= NBUF:
                    writes[f - NBUF].wait()
                gathers[f] = issue_gather(f)
            gathers[j].wait()
            writes[j] = issue_write(j)
        for j in range(max(0, nchunk - NBUF), nchunk):
            writes[j].wait()

    return k(idx3, weight)


def kernel(x, weight):
    dim = weight.shape[1]
    b_total = x.size
    nchunk = b_total // (NW * CHUNK)
    out = _gather_sc(x.reshape(-1), weight, nchunk, dim)
    return out.reshape(x.shape + (dim,))
